# baseline (speedup 1.0000x reference)
import functools

import jax
import jax.numpy as jnp
from jax import lax
from jax.experimental import pallas as pl
from jax.experimental.pallas import tpu as pltpu
from jax.experimental.pallas import tpu_sc as plsc

VOCAB = 1000000
D = 64
NC, NS = 2, 16
NW = NC * NS
G = 128
NF = 8
CHUNK = NF * G


def _make_kernel(B):
    assert B % (NW * CHUNK) == 0
    b_per_w = B // NW
    n_iter = b_per_w // CHUNK
    mesh = plsc.VectorSubcoreMesh(core_axis_name="c", subcore_axis_name="s")

    @functools.partial(
        pl.kernel,
        mesh=mesh,
        out_type=jax.ShapeDtypeStruct((B, D), jnp.float32),
        scratch_types=[
            pltpu.VMEM((NF, G), jnp.int32),
            pltpu.VMEM((CHUNK, D), jnp.float32),
            pltpu.SemaphoreType.DMA,
        ],
        compiler_params=pltpu.CompilerParams(use_tc_tiling_on_sc=False),
    )
    def k(table_hbm, idx_hbm, out_hbm, idx_v, rows_v, sem):
        wid = lax.axis_index("s") * NC + lax.axis_index("c")
        base = wid * n_iter

        def body(i, carry):
            row = base + i
            pltpu.sync_copy(idx_hbm.at[pl.ds(row * NF, NF)], idx_v)
            for j in range(NF):
                pltpu.async_copy(
                    table_hbm.at[idx_v.at[j]],
                    rows_v.at[pl.ds(j * G, G)],
                    sem,
                )
            pltpu.make_async_copy(
                table_hbm.at[pl.ds(0, CHUNK)], rows_v, sem
            ).wait()
            pltpu.sync_copy(
                rows_v, out_hbm.at[pl.ds((row * NF) * G, CHUNK)]
            )
            return carry

        lax.fori_loop(0, n_iter, body, 0)

    return k


def kernel(indices, weight):
    B = indices.shape[0] * indices.shape[1]
    idx = jnp.reshape(indices.astype(jnp.int32), (B // G, G))
    out = _make_kernel(B)(weight, idx)
    return jnp.reshape(out, (*indices.shape, D))


# speedup vs baseline: 1.0166x; 1.0166x over previous
import functools

import jax
import jax.numpy as jnp
from jax import lax
from jax.experimental import pallas as pl
from jax.experimental.pallas import tpu as pltpu
from jax.experimental.pallas import tpu_sc as plsc

D = 64
NC, NS = 2, 16
NW = NC * NS
G = 128
NF = 2
CHUNK = NF * G
NBUF = 4
LA = 2


def _make_kernel(B):
    b_per_w = B // NW
    n_iter = b_per_w // CHUNK
    assert b_per_w % CHUNK == 0 and (n_iter - NBUF) % NBUF == 0
    n_outer = (n_iter - NBUF) // NBUF
    idx_rows = n_iter * NF
    mesh = plsc.VectorSubcoreMesh(core_axis_name="c", subcore_axis_name="s")

    @functools.partial(
        pl.kernel,
        mesh=mesh,
        out_type=jax.ShapeDtypeStruct((B, D), jnp.float32),
        scratch_types=[
            pltpu.VMEM((idx_rows, G), jnp.int32),
            pltpu.VMEM((NBUF * CHUNK, D), jnp.float32),
        ]
        + [pltpu.SemaphoreType.DMA] * (2 * NBUF),
        compiler_params=pltpu.CompilerParams(use_tc_tiling_on_sc=False),
    )
    def k(table, idx_hbm, out_hbm, idx_all, rows, *sems):
        gsem, osem = sems[:NBUF], sems[NBUF:]
        wid = lax.axis_index("s") * NC + lax.axis_index("c")
        pltpu.sync_copy(idx_hbm.at[pl.ds(wid * idx_rows, idx_rows)], idx_all)
        out_base = wid * b_per_w

        def fire(c, b):
            for j in range(NF):
                pltpu.async_copy(
                    table.at[idx_all.at[c * NF + j]],
                    rows.at[pl.ds(b * CHUNK + j * G, G)],
                    gsem[b],
                )

        def drain(b):
            pltpu.make_async_copy(
                table.at[pl.ds(0, CHUNK)],
                rows.at[pl.ds(b * CHUNK, CHUNK)],
                gsem[b],
            ).wait()

        def write(c, b):
            pltpu.async_copy(
                rows.at[pl.ds(b * CHUNK, CHUNK)],
                out_hbm.at[pl.ds(out_base + c * CHUNK, CHUNK)],
                osem[b],
            )

        def wait_write(b):
            pltpu.make_async_copy(
                rows.at[pl.ds(b * CHUNK, CHUNK)],
                out_hbm.at[pl.ds(0, CHUNK)],
                osem[b],
            ).wait()

        fire(0, 0)
        fire(1, 1)
        fire(2, 2)
        drain(0)
        write(0, 0)
        fire(3, 3)
        drain(1)
        write(1, 1)

        def body(g, carry):
            c0 = NBUF + g * NBUF
            for b in range(NBUF):
                c = c0 + b
                wait_write(b)
                fire(c, b)
                bd = (b + NBUF - LA) % NBUF
                drain(bd)
                write(c - LA, bd)
            return carry

        lax.fori_loop(0, n_outer, body, 0)

        for c in range(n_iter - LA, n_iter):
            b = c % NBUF
            drain(b)
            write(c, b)
        for b in range(NBUF):
            wait_write(b)

    return k


def kernel(indices, weight):
    B = indices.shape[0] * indices.shape[1]
    idx = jnp.reshape(indices.astype(jnp.int32), (B // G, G))
    out = _make_kernel(B)(weight, idx)
    return jnp.reshape(out, (*indices.shape, D))
